# baseline (speedup 1.0000x reference)
import functools

import jax
import jax.numpy as jnp
from jax import lax
from jax.experimental import pallas as pl
from jax.experimental.pallas import tpu as pltpu
from jax.experimental.pallas import tpu_sc as plsc

F32 = jnp.float32

N = 10000
NP_ = 10000
E = 320000
NG = 16
DH = 128

NC = 2
NS = 16
L = 16
NW = NC * NS
EPW = E // NW
CH = 80
NCH = EPW // CH
RPS = 632
RPS_LAST = N - RPS * (NS - 1)

NB = 2000
EB = 4000
NBLK = E // EB


def _full(shape):
    return pl.BlockSpec(shape, lambda i: tuple(0 for _ in shape))


def _mesh():
    return plsc.VectorSubcoreMesh(
        core_axis_name="c", subcore_axis_name="s", num_cores=NC, num_subcores=NS
    )


def _pool_body(hp_ref, mk_ref, W1, b1, W2, b2, out_ref, acc, cnt):
    i = pl.program_id(0)

    @pl.when(i == 0)
    def _():
        acc[...] = jnp.zeros_like(acc)
        cnt[...] = jnp.zeros_like(cnt)

    z = jax.nn.silu(
        jnp.dot(hp_ref[...], W1[...], preferred_element_type=F32) + b1[...]
    )
    z = jnp.dot(z, W2[...], preferred_element_type=F32) + b2[...]
    gid = lax.broadcasted_iota(jnp.int32, (NB, NG), 1).astype(F32)
    onehot = (mk_ref[...] == gid).astype(F32)
    dn = (((0,), (0,)), ((), ()))
    acc[...] += lax.dot_general(onehot, z, dn, preferred_element_type=F32)
    ones = jnp.ones((NB, 1), F32)
    cnt[...] += lax.dot_general(onehot, ones, dn, preferred_element_type=F32)

    @pl.when(i == pl.num_programs(0) - 1)
    def _():
        out_ref[...] = acc[...] / jnp.maximum(cnt[...], 1.0)


def _pool_call(pocket_h, maskp_f, W1, b1, W2, b2):
    return pl.pallas_call(
        _pool_body,
        grid=(NP_ // NB,),
        in_specs=[
            pl.BlockSpec((NB, DH), lambda i: (i, 0)),
            pl.BlockSpec((NB, 1), lambda i: (i, 0)),
            _full(W1.shape), _full(b1.shape), _full(W2.shape), _full(b2.shape),
        ],
        out_specs=_full((NG, DH)),
        out_shape=jax.ShapeDtypeStruct((NG, DH), F32),
        scratch_shapes=[pltpu.VMEM((NG, DH), F32), pltpu.VMEM((NG, 1), F32)],
    )(pocket_h, maskp_f, W1, b1, W2, b2)


def _node_body(ha_ref, mk_ref, pool_ref, t_ref,
               W1, b1, W2, b2, Wsrc, Wdst, msg_b,
               h_ref, ap_ref, bp_ref):
    z = jax.nn.silu(
        jnp.dot(ha_ref[...], W1[...], preferred_element_type=F32) + b1[...]
    )
    z = jnp.dot(z, W2[...], preferred_element_type=F32) + b2[...]
    gid = lax.broadcasted_iota(jnp.int32, (NB, NG), 1).astype(F32)
    onehot = (mk_ref[...] == gid).astype(F32)
    z = z + jnp.dot(onehot, t_ref[...], preferred_element_type=F32)
    z = z + jnp.dot(onehot, pool_ref[...], preferred_element_type=F32)
    h_ref[...] = z
    ap_ref[...] = jnp.dot(z, Wsrc[...], preferred_element_type=F32)
    bp_ref[...] = jnp.dot(z, Wdst[...], preferred_element_type=F32) + msg_b[...]


def _node_call(h_atoms, maska_f, pool, t, W1, b1, W2, b2, Wsrc, Wdst, msg_b):
    out_shape = tuple(jax.ShapeDtypeStruct((N, DH), F32) for _ in range(3))
    return pl.pallas_call(
        _node_body,
        grid=(N // NB,),
        in_specs=[
            pl.BlockSpec((NB, DH), lambda i: (i, 0)),
            pl.BlockSpec((NB, 1), lambda i: (i, 0)),
            _full((NG, DH)), _full((NG, 1)),
            _full(W1.shape), _full(b1.shape), _full(W2.shape), _full(b2.shape),
            _full(Wsrc.shape), _full(Wdst.shape), _full(msg_b.shape),
        ],
        out_specs=tuple(pl.BlockSpec((NB, DH), lambda i: (i, 0)) for _ in range(3)),
        out_shape=out_shape,
    )(h_atoms, maska_f, pool, t, W1, b1, W2, b2, Wsrc, Wdst, msg_b)


def _gather_body(ap_hbm, bp_hbm, xx_hbm, xy_hbm, xz_hbm, src_hbm, dst_hbm,
                 s_hbm, d2_hbm, dx_hbm, dy_hbm, dz_hbm,
                 si0, si1, di0, di1, ra0, ra1, rb0, rb1,
                 d20, d21, dx0, dx1, dy0, dy1, dz0, dz1,
                 xx_v, xy_v, xz_v,
                 is0, is1, gs0, gs1, ws0, ws1):
    SI = (si0, si1)
    DI = (di0, di1)
    RA = (ra0, ra1)
    RB = (rb0, rb1)
    D2 = (d20, d21)
    DX = (dx0, dx1)
    DY = (dy0, dy1)
    DZ = (dz0, dz1)
    IS = (is0, is1)
    GS = (gs0, gs1)
    WS = (ws0, ws1)
    wid = lax.axis_index("s") * NC + lax.axis_index("c")
    base = wid * EPW

    def issue_idx(b, c):
        off = base + c * CH
        pltpu.async_copy(src_hbm.at[pl.ds(off, CH)], SI[b], IS[b])
        pltpu.async_copy(dst_hbm.at[pl.ds(off, CH)], DI[b], IS[b])

    def wait_idx(b):
        pltpu.make_async_copy(src_hbm.at[pl.ds(0, CH)], SI[b], IS[b]).wait()
        pltpu.make_async_copy(src_hbm.at[pl.ds(0, CH)], DI[b], IS[b]).wait()

    def issue_gather(b):
        pltpu.async_copy(ap_hbm.at[SI[b]], RA[b], GS[b])
        pltpu.async_copy(bp_hbm.at[DI[b]], RB[b], GS[b])

    def wait_gather(b):
        pltpu.make_async_copy(ap_hbm.at[pl.ds(0, CH)], RA[b], GS[b]).wait()
        pltpu.make_async_copy(bp_hbm.at[pl.ds(0, CH)], RB[b], GS[b]).wait()

    def issue_writes(b, c):
        off = base + c * CH
        pltpu.async_copy(RA[b], s_hbm.at[pl.ds(off, CH)], WS[b])
        pltpu.async_copy(D2[b], d2_hbm.at[pl.ds(off, CH)], WS[b])
        pltpu.async_copy(DX[b], dx_hbm.at[pl.ds(off, CH)], WS[b])
        pltpu.async_copy(DY[b], dy_hbm.at[pl.ds(off, CH)], WS[b])
        pltpu.async_copy(DZ[b], dz_hbm.at[pl.ds(off, CH)], WS[b])

    def wait_writes(b):
        pltpu.make_async_copy(RA[b], s_hbm.at[pl.ds(0, CH)], WS[b]).wait()
        pltpu.make_async_copy(D2[b], d2_hbm.at[pl.ds(0, CH)], WS[b]).wait()
        pltpu.make_async_copy(DX[b], dx_hbm.at[pl.ds(0, CH)], WS[b]).wait()
        pltpu.make_async_copy(DY[b], dy_hbm.at[pl.ds(0, CH)], WS[b]).wait()
        pltpu.make_async_copy(DZ[b], dz_hbm.at[pl.ds(0, CH)], WS[b]).wait()

    def dist_loop(b):
        def dist(k, kcarry):
            sl = pl.ds(k * L, L)
            ivs = SI[b][sl]
            ivd = DI[b][sl]
            dx = plsc.load_gather(xx_v, [ivd]) - plsc.load_gather(xx_v, [ivs])
            dy = plsc.load_gather(xy_v, [ivd]) - plsc.load_gather(xy_v, [ivs])
            dz = plsc.load_gather(xz_v, [ivd]) - plsc.load_gather(xz_v, [ivs])
            DX[b][sl] = dx
            DY[b][sl] = dy
            DZ[b][sl] = dz
            D2[b][sl] = dx * dx + dy * dy + dz * dz
            return kcarry

        lax.fori_loop(0, CH // L, dist, 0, unroll=True)

    def rowadd_loop(b):
        def row(i, rcarry):
            for j in range(DH // L):
                sl = pl.ds(j * L, L)
                RA[b][i, sl] = RA[b][i, sl] + RB[b][i, sl]
            return rcarry

        lax.fori_loop(0, CH, row, 0, unroll=4)

    pltpu.sync_copy(xx_hbm, xx_v)
    pltpu.sync_copy(xy_hbm, xy_v)
    pltpu.sync_copy(xz_hbm, xz_v)
    issue_idx(0, 0)
    wait_idx(0)
    issue_gather(0)

    def body(i, carry):
        c0 = 2 * i
        issue_idx(1, c0 + 1)
        dist_loop(0)
        wait_idx(1)

        @pl.when(i > 0)
        def _():
            wait_writes(1)

        issue_gather(1)
        wait_gather(0)
        rowadd_loop(0)
        issue_writes(0, c0)

        issue_idx(0, c0 + 2)
        dist_loop(1)
        wait_idx(0)
        wait_writes(0)
        issue_gather(0)
        wait_gather(1)
        rowadd_loop(1)
        issue_writes(1, c0 + 1)
        return carry

    lax.fori_loop(0, (NCH - 1) // 2, body, 0, unroll=False)
    dist_loop(0)
    wait_gather(0)
    rowadd_loop(0)
    issue_writes(0, NCH - 1)
    wait_writes(0)
    wait_writes(1)


def _gather_call(ap, bp, xx, xy, xz, src, dst):
    k = functools.partial(
        pl.kernel,
        out_type=(
            jax.ShapeDtypeStruct((E, DH), F32),
            jax.ShapeDtypeStruct((E,), F32),
            jax.ShapeDtypeStruct((E,), F32),
            jax.ShapeDtypeStruct((E,), F32),
            jax.ShapeDtypeStruct((E,), F32),
        ),
        mesh=_mesh(),
        scratch_types=(
            [pltpu.VMEM((CH,), jnp.int32) for _ in range(4)]
            + [pltpu.VMEM((CH, DH), F32) for _ in range(4)]
            + [pltpu.VMEM((CH,), F32) for _ in range(8)]
            + [pltpu.VMEM((N,), F32) for _ in range(3)]
            + [pltpu.SemaphoreType.DMA for _ in range(6)]
        ),
        compiler_params=pltpu.CompilerParams(needs_layout_passes=False),
    )(_gather_body)
    return k(ap, bp, xx, xy, xz, src, dst)


def _edge_body(s_ref, d2_ref, bt_ref, bW1, bb1, bW2, bb2, We, wc, cW,
               eW1, eb1, eW2, eb2, msg_ref, le_ref, sc_ref):
    dn0 = (((0,), (0,)), ((), ()))
    bt = bt_ref[0]
    e = jax.nn.silu(
        lax.dot_general(bt, bW1[...], dn0, preferred_element_type=F32) + bb1[...]
    )
    e = jnp.dot(e, bW2[...], preferred_element_type=F32) + bb2[...]
    d2row = d2_ref[0]
    d2wc = lax.dot_general(d2row, wc[...], dn0, preferred_element_type=F32)
    msg = jax.nn.silu(
        s_ref[...] + d2wc + jnp.dot(e, We[...], preferred_element_type=F32)
    )
    msg_ref[...] = msg
    z2 = jax.nn.silu(
        jnp.dot(msg, eW1[...], preferred_element_type=F32) + eb1[...]
    )
    le_ref[...] = jnp.dot(z2, eW2[...], preferred_element_type=F32) + eb2[...]
    dn_cw = (((0,), (1,)), ((), ()))
    sc_ref[0] = lax.dot_general(cW[...], msg, dn_cw, preferred_element_type=F32)


def _edge_call(S, d23, bt3, bW1, bb1, bW2, bb2, We, wc, cW, eW1, eb1, eW2, eb2):
    out_shape = (
        jax.ShapeDtypeStruct((E, DH), F32),
        jax.ShapeDtypeStruct((E, 5), F32),
        jax.ShapeDtypeStruct((NBLK, 1, EB), F32),
    )
    return pl.pallas_call(
        _edge_body,
        grid=(NBLK,),
        in_specs=[
            pl.BlockSpec((EB, DH), lambda i: (i, 0)),
            pl.BlockSpec((1, 1, EB), lambda i: (i, 0, 0)),
            pl.BlockSpec((1, 5, EB), lambda i: (i, 0, 0)),
            _full(bW1.shape), _full(bb1.shape), _full(bW2.shape), _full(bb2.shape),
            _full(We.shape), _full(wc.shape), _full(cW.shape),
            _full(eW1.shape), _full(eb1.shape), _full(eW2.shape), _full(eb2.shape),
        ],
        out_specs=(
            pl.BlockSpec((EB, DH), lambda i: (i, 0)),
            pl.BlockSpec((EB, 5), lambda i: (i, 0)),
            pl.BlockSpec((1, 1, EB), lambda i: (i, 0, 0)),
        ),
        out_shape=out_shape,
    )(S, d23, bt3, bW1, bb1, bW2, bb2, We, wc, cW, eW1, eb1, eW2, eb2)


def _aggscatter_body(msg_hbm, dst_hbm, z128_hbm, agg_hbm,
                     di0, di1, di2, m0, m1, m2,
                     ls0, ls1, ls2, ss0, ss1, ss2, agg_s):
    DI = (di0, di1, di2)
    M = (m0, m1, m2)
    LS = (ls0, ls1, ls2)
    SS = (ss0, ss1, ss2)
    c = lax.axis_index("c")
    s = lax.axis_index("s")
    wid = c * NS + s

    @pl.when(s < NS - 1)
    def _():
        sl = pl.ds(s * RPS, RPS)
        pltpu.sync_copy(z128_hbm.at[sl], agg_s.at[sl])

    @pl.when(s == NS - 1)
    def _():
        sl = pl.ds(RPS * (NS - 1), RPS_LAST)
        pltpu.sync_copy(z128_hbm.at[sl], agg_s.at[sl])

    plsc.subcore_barrier()
    base = wid * EPW

    def issue_loads(b, ci):
        off = base + ci * CH
        pltpu.async_copy(dst_hbm.at[pl.ds(off, CH)], DI[b], LS[b])
        pltpu.async_copy(msg_hbm.at[pl.ds(off, CH)], M[b], LS[b])

    def wait_loads(b):
        pltpu.make_async_copy(dst_hbm.at[pl.ds(0, CH)], DI[b], LS[b]).wait()
        pltpu.make_async_copy(msg_hbm.at[pl.ds(0, CH)], M[b], LS[b]).wait()

    def issue_scatter(b):
        pltpu.async_copy(M[b], agg_s.at[DI[b]], SS[b], add=True)

    def wait_scatter(b):
        pltpu.make_async_copy(M[b], agg_s.at[pl.ds(0, CH)], SS[b]).wait()

    issue_loads(0, 0)
    issue_loads(1, 1)

    def body(i, carry):
        c = 3 * i
        wait_loads(0)
        issue_scatter(0)

        @pl.when(i > 0)
        def _():
            wait_scatter(2)

        issue_loads(2, c + 2)
        wait_loads(1)
        issue_scatter(1)
        wait_scatter(0)
        issue_loads(0, c + 3)
        wait_loads(2)
        issue_scatter(2)
        wait_scatter(1)
        issue_loads(1, c + 4)
        return carry

    lax.fori_loop(0, (NCH - 2) // 3, body, 0, unroll=False)
    wait_scatter(2)
    wait_loads(0)
    issue_scatter(0)
    wait_loads(1)
    issue_scatter(1)
    wait_scatter(0)
    wait_scatter(1)
    plsc.subcore_barrier()

    @pl.when(s < NS - 1)
    def _():
        sl = pl.ds(s * RPS, RPS)
        pltpu.sync_copy(agg_s.at[sl], agg_hbm.at[c, sl])

    @pl.when(s == NS - 1)
    def _():
        sl = pl.ds(RPS * (NS - 1), RPS_LAST)
        pltpu.sync_copy(agg_s.at[sl], agg_hbm.at[c, sl])


def _velscatter_body(scal_hbm, dst_hbm, dx_hbm, dy_hbm, dz_hbm, zn_hbm,
                     vx_hbm, vy_hbm, vz_hbm,
                     di0, di1, sc0, sc1, dxc0, dxc1, dyc0, dyc1, dzc0, dzc1,
                     vx_v, vy_v, vz_v, ls0, ls1):
    DI = (di0, di1)
    SC = (sc0, sc1)
    DXC = (dxc0, dxc1)
    DYC = (dyc0, dyc1)
    DZC = (dzc0, dzc1)
    LS = (ls0, ls1)
    c = lax.axis_index("c")
    s = lax.axis_index("s")
    wid = c * NS + s
    pltpu.sync_copy(zn_hbm, vx_v)
    pltpu.sync_copy(zn_hbm, vy_v)
    pltpu.sync_copy(zn_hbm, vz_v)
    base = wid * EPW

    def issue_loads(b, ci):
        off = base + ci * CH
        pltpu.async_copy(dst_hbm.at[pl.ds(off, CH)], DI[b], LS[b])
        pltpu.async_copy(scal_hbm.at[pl.ds(off, CH)], SC[b], LS[b])
        pltpu.async_copy(dx_hbm.at[pl.ds(off, CH)], DXC[b], LS[b])
        pltpu.async_copy(dy_hbm.at[pl.ds(off, CH)], DYC[b], LS[b])
        pltpu.async_copy(dz_hbm.at[pl.ds(off, CH)], DZC[b], LS[b])

    def wait_loads(b):
        pltpu.make_async_copy(dst_hbm.at[pl.ds(0, CH)], DI[b], LS[b]).wait()
        pltpu.make_async_copy(scal_hbm.at[pl.ds(0, CH)], SC[b], LS[b]).wait()
        pltpu.make_async_copy(dx_hbm.at[pl.ds(0, CH)], DXC[b], LS[b]).wait()
        pltpu.make_async_copy(dy_hbm.at[pl.ds(0, CH)], DYC[b], LS[b]).wait()
        pltpu.make_async_copy(dz_hbm.at[pl.ds(0, CH)], DZC[b], LS[b]).wait()

    def vel_loop(b):
        def vel(k, kcarry):
            ksl = pl.ds(k * L, L)
            ivd = DI[b][ksl]
            scv = SC[b][ksl]
            plsc.addupdate_scatter(vx_v, [ivd], DXC[b][ksl] * scv)
            plsc.addupdate_scatter(vy_v, [ivd], DYC[b][ksl] * scv)
            plsc.addupdate_scatter(vz_v, [ivd], DZC[b][ksl] * scv)
            return kcarry

        lax.fori_loop(0, CH // L, vel, 0, unroll=True)

    issue_loads(0, 0)

    def body(i, carry):
        c0 = 2 * i
        issue_loads(1, c0 + 1)
        wait_loads(0)
        vel_loop(0)
        issue_loads(0, c0 + 2)
        wait_loads(1)
        vel_loop(1)
        return carry

    lax.fori_loop(0, (NCH - 1) // 2, body, 0, unroll=False)
    wait_loads(0)
    vel_loop(0)
    pltpu.sync_copy(vx_v, vx_hbm.at[wid, 0])
    pltpu.sync_copy(vy_v, vy_hbm.at[wid, 0])
    pltpu.sync_copy(vz_v, vz_hbm.at[wid, 0])


def _scatter_call(msg, scal, dst, dx, dy, dz):
    z128 = jnp.zeros((N, DH), F32)
    zn = jnp.zeros((N,), F32)
    ka = functools.partial(
        pl.kernel,
        out_type=jax.ShapeDtypeStruct((NC, N, DH), F32),
        mesh=_mesh(),
        scratch_types=(
            [pltpu.VMEM((CH,), jnp.int32) for _ in range(3)]
            + [pltpu.VMEM((CH, DH), F32) for _ in range(3)]
            + [pltpu.SemaphoreType.DMA for _ in range(6)]
            + [pltpu.VMEM_SHARED((N, DH), F32)]
        ),
        compiler_params=pltpu.CompilerParams(needs_layout_passes=False),
    )(_aggscatter_body)
    aggp = ka(msg, dst, z128)
    kv = functools.partial(
        pl.kernel,
        out_type=(
            jax.ShapeDtypeStruct((NW, 1, N), F32),
            jax.ShapeDtypeStruct((NW, 1, N), F32),
            jax.ShapeDtypeStruct((NW, 1, N), F32),
        ),
        mesh=_mesh(),
        scratch_types=(
            [pltpu.VMEM((CH,), jnp.int32) for _ in range(2)]
            + [pltpu.VMEM((CH,), F32) for _ in range(8)]
            + [pltpu.VMEM((N,), F32) for _ in range(3)]
            + [pltpu.SemaphoreType.DMA for _ in range(2)]
        ),
        compiler_params=pltpu.CompilerParams(needs_layout_passes=False),
    )(_velscatter_body)
    vx, vy, vz = kv(scal, dst, dx, dy, dz, zn)
    return aggp, vx, vy, vz


def _dec_body(h_ref, agg_ref, vx_ref, vy_ref, vz_ref, updW, W1, b1, W2, b2,
              vel_ref, lh_ref):
    agg = agg_ref[0] + agg_ref[1]
    h_out = h_ref[...] + jnp.dot(agg, updW[...], preferred_element_type=F32)
    z = jax.nn.silu(
        jnp.dot(h_out, W1[...], preferred_element_type=F32) + b1[...]
    )
    lh_ref[...] = jnp.dot(z, W2[...], preferred_element_type=F32) + b2[...]
    cx = jnp.sum(vx_ref[...], axis=1, keepdims=True)
    cy = jnp.sum(vy_ref[...], axis=1, keepdims=True)
    cz = jnp.sum(vz_ref[...], axis=1, keepdims=True)
    vel_ref[...] = jnp.concatenate([cx, cy, cz], axis=1)


def _dec_call(h, aggp, vxT, vyT, vzT, updW, W1, b1, W2, b2):
    out_shape = (
        jax.ShapeDtypeStruct((N, 3), F32),
        jax.ShapeDtypeStruct((N, DH), F32),
    )
    return pl.pallas_call(
        _dec_body,
        grid=(N // NB,),
        in_specs=[
            pl.BlockSpec((NB, DH), lambda i: (i, 0)),
            pl.BlockSpec((NC, NB, DH), lambda i: (0, i, 0)),
            pl.BlockSpec((NB, NW), lambda i: (i, 0)),
            pl.BlockSpec((NB, NW), lambda i: (i, 0)),
            pl.BlockSpec((NB, NW), lambda i: (i, 0)),
            _full(updW.shape),
            _full(W1.shape), _full(b1.shape), _full(W2.shape), _full(b2.shape),
        ],
        out_specs=(
            pl.BlockSpec((NB, 3), lambda i: (i, 0)),
            pl.BlockSpec((NB, DH), lambda i: (i, 0)),
        ),
        out_shape=out_shape,
    )(h, aggp, vxT, vyT, vzT, updW, W1, b1, W2, b2)


def kernel(x_atoms, h_atoms, rot_vec, mask_atoms, pocket_h, mask_pocket, t,
           bond_index, bond_types, params):
    p = params
    r1 = lambda b: b.reshape(1, -1)

    maskp_f = mask_pocket.astype(F32).reshape(NP_, 1)
    maska_f = mask_atoms.astype(F32).reshape(N, 1)
    xx = x_atoms[:, 0]
    xy = x_atoms[:, 1]
    xz = x_atoms[:, 2]
    src = bond_index[0].astype(jnp.int32)
    dst = bond_index[1].astype(jnp.int32)
    bt3 = bond_types.reshape(NBLK, EB, 5).transpose(0, 2, 1)

    msg_W = p['msg_W']
    Wsrc = msg_W[0:DH]
    Wdst = msg_W[DH:2 * DH]
    wc = msg_W[2 * DH:2 * DH + 1]
    We = msg_W[2 * DH + 1:]

    pool = _pool_call(pocket_h, maskp_f,
                      p['re_W1'], r1(p['re_b1']), p['re_W2'], r1(p['re_b2']))
    h, ap, bp = _node_call(h_atoms, maska_f, pool, t,
                           p['ae_W1'], r1(p['ae_b1']), p['ae_W2'], r1(p['ae_b2']),
                           Wsrc, Wdst, r1(p['msg_b']))
    S, d2, dx, dy, dz = _gather_call(ap, bp, xx, xy, xz, src, dst)
    msg, logits_e, scal3 = _edge_call(
        S, d2.reshape(NBLK, 1, EB), bt3,
        p['be_W1'], r1(p['be_b1']), p['be_W2'], r1(p['be_b2']),
        We, wc, p['cW'],
        p['ed_W1'], r1(p['ed_b1']), p['ed_W2'], r1(p['ed_b2']))
    aggp, vx, vy, vz = _scatter_call(msg, scal3.reshape(E), dst, dx, dy, dz)
    vx = vx.reshape(NW, N).T
    vy = vy.reshape(NW, N).T
    vz = vz.reshape(NW, N).T
    vel, logits_h = _dec_call(h, aggp, vx, vy, vz, p['upd_W'],
                              p['ad_W1'], r1(p['ad_b1']),
                              p['ad_W2'], r1(p['ad_b2']))
    return vel, logits_h, logits_e


# speedup vs baseline: 1.3773x; 1.3773x over previous
import functools

import jax
import jax.numpy as jnp
from jax import lax
from jax.experimental import pallas as pl
from jax.experimental.pallas import tpu as pltpu
from jax.experimental.pallas import tpu_sc as plsc

F32 = jnp.float32

N = 10000
NP_ = 10000
E = 320000
NG = 16
DH = 128

NC = 2
NS = 16
L = 16
NW = NC * NS
EPW = E // NW
CH = 80
NCH = EPW // CH
RPS = 632
RPS_LAST = N - RPS * (NS - 1)

NB = 2000
EB = 8000
NBLK = E // EB


def _full(shape):
    return pl.BlockSpec(shape, lambda i: tuple(0 for _ in shape))


def _mesh():
    return plsc.VectorSubcoreMesh(
        core_axis_name="c", subcore_axis_name="s", num_cores=NC, num_subcores=NS
    )


def _pool_body(hp_ref, mk_ref, W1, b1, W2, b2, out_ref, acc, cnt):
    i = pl.program_id(0)

    @pl.when(i == 0)
    def _():
        acc[...] = jnp.zeros_like(acc)
        cnt[...] = jnp.zeros_like(cnt)

    z = jax.nn.silu(
        jnp.dot(hp_ref[...], W1[...], preferred_element_type=F32) + b1[...]
    )
    z = jnp.dot(z, W2[...], preferred_element_type=F32) + b2[...]
    gid = lax.broadcasted_iota(jnp.int32, (NB, NG), 1).astype(F32)
    onehot = (mk_ref[...] == gid).astype(F32)
    dn = (((0,), (0,)), ((), ()))
    acc[...] += lax.dot_general(onehot, z, dn, preferred_element_type=F32)
    ones = jnp.ones((NB, 1), F32)
    cnt[...] += lax.dot_general(onehot, ones, dn, preferred_element_type=F32)

    @pl.when(i == pl.num_programs(0) - 1)
    def _():
        out_ref[...] = acc[...] / jnp.maximum(cnt[...], 1.0)


def _pool_call(pocket_h, maskp_f, W1, b1, W2, b2):
    return pl.pallas_call(
        _pool_body,
        grid=(NP_ // NB,),
        in_specs=[
            pl.BlockSpec((NB, DH), lambda i: (i, 0)),
            pl.BlockSpec((NB, 1), lambda i: (i, 0)),
            _full(W1.shape), _full(b1.shape), _full(W2.shape), _full(b2.shape),
        ],
        out_specs=_full((NG, DH)),
        out_shape=jax.ShapeDtypeStruct((NG, DH), F32),
        scratch_shapes=[pltpu.VMEM((NG, DH), F32), pltpu.VMEM((NG, 1), F32)],
    )(pocket_h, maskp_f, W1, b1, W2, b2)


def _node_body(ha_ref, mk_ref, pool_ref, t_ref,
               W1, b1, W2, b2, Wsrc, Wdst, msg_b,
               h_ref, ap_ref, bp_ref):
    z = jax.nn.silu(
        jnp.dot(ha_ref[...], W1[...], preferred_element_type=F32) + b1[...]
    )
    z = jnp.dot(z, W2[...], preferred_element_type=F32) + b2[...]
    gid = lax.broadcasted_iota(jnp.int32, (NB, NG), 1).astype(F32)
    onehot = (mk_ref[...] == gid).astype(F32)
    z = z + jnp.dot(onehot, t_ref[...], preferred_element_type=F32)
    z = z + jnp.dot(onehot, pool_ref[...], preferred_element_type=F32)
    h_ref[...] = z
    ap_ref[...] = jnp.dot(z, Wsrc[...], preferred_element_type=F32)
    bp_ref[...] = jnp.dot(z, Wdst[...], preferred_element_type=F32) + msg_b[...]


def _node_call(h_atoms, maska_f, pool, t, W1, b1, W2, b2, Wsrc, Wdst, msg_b):
    out_shape = tuple(jax.ShapeDtypeStruct((N, DH), F32) for _ in range(3))
    return pl.pallas_call(
        _node_body,
        grid=(N // NB,),
        in_specs=[
            pl.BlockSpec((NB, DH), lambda i: (i, 0)),
            pl.BlockSpec((NB, 1), lambda i: (i, 0)),
            _full((NG, DH)), _full((NG, 1)),
            _full(W1.shape), _full(b1.shape), _full(W2.shape), _full(b2.shape),
            _full(Wsrc.shape), _full(Wdst.shape), _full(msg_b.shape),
        ],
        out_specs=tuple(pl.BlockSpec((NB, DH), lambda i: (i, 0)) for _ in range(3)),
        out_shape=out_shape,
    )(h_atoms, maska_f, pool, t, W1, b1, W2, b2, Wsrc, Wdst, msg_b)


def _gather_body(ap_hbm, bp_hbm, xx_hbm, xy_hbm, xz_hbm, src_hbm, dst_hbm,
                 s_hbm, d2_hbm, dt_hbm,
                 si0, si1, si2, di0, di1, di2, ra0, ra1, ra2,
                 rb0, rb1, rb2, d20, d21, d22, dt0, dt1, dt2,
                 xx_v, xy_v, xz_v,
                 is0, is1, is2, gs0, gs1, gs2, ws0, ws1, ws2):
    SI = (si0, si1, si2)
    DI = (di0, di1, di2)
    RA = (ra0, ra1, ra2)
    RB = (rb0, rb1, rb2)
    D2 = (d20, d21, d22)
    DT = (dt0, dt1, dt2)
    IS = (is0, is1, is2)
    GS = (gs0, gs1, gs2)
    WS = (ws0, ws1, ws2)
    wid = lax.axis_index("s") * NC + lax.axis_index("c")
    base = wid * EPW

    def issue_idx(b, c):
        off = base + c * CH
        pltpu.async_copy(src_hbm.at[pl.ds(off, CH)], SI[b], IS[b])
        pltpu.async_copy(dst_hbm.at[pl.ds(off, CH)], DI[b], IS[b])

    def wait_idx(b):
        pltpu.make_async_copy(src_hbm.at[pl.ds(0, CH)], SI[b], IS[b]).wait()
        pltpu.make_async_copy(src_hbm.at[pl.ds(0, CH)], DI[b], IS[b]).wait()

    def issue_gather(b):
        pltpu.async_copy(ap_hbm.at[SI[b]], RA[b], GS[b])
        pltpu.async_copy(bp_hbm.at[DI[b]], RB[b], GS[b])

    def wait_gather(b):
        pltpu.make_async_copy(ap_hbm.at[pl.ds(0, CH)], RA[b], GS[b]).wait()
        pltpu.make_async_copy(bp_hbm.at[pl.ds(0, CH)], RB[b], GS[b]).wait()

    def issue_writes(b, c):
        off = base + c * CH
        pltpu.async_copy(RA[b], s_hbm.at[pl.ds(off, CH)], WS[b])
        pltpu.async_copy(D2[b], d2_hbm.at[pl.ds(off, CH)], WS[b])
        pltpu.async_copy(DT[b], dt_hbm.at[pl.ds(3 * off, 3 * CH)], WS[b])

    def wait_writes(b):
        pltpu.make_async_copy(RA[b], s_hbm.at[pl.ds(0, CH)], WS[b]).wait()
        pltpu.make_async_copy(D2[b], d2_hbm.at[pl.ds(0, CH)], WS[b]).wait()
        pltpu.make_async_copy(DT[b], dt_hbm.at[pl.ds(0, 3 * CH)], WS[b]).wait()

    def dist_loop(b):
        def dist(k, kcarry):
            sl = pl.ds(k * L, L)
            ivs = SI[b][sl]
            ivd = DI[b][sl]
            dx = plsc.load_gather(xx_v, [ivd]) - plsc.load_gather(xx_v, [ivs])
            dy = plsc.load_gather(xy_v, [ivd]) - plsc.load_gather(xy_v, [ivs])
            dz = plsc.load_gather(xz_v, [ivd]) - plsc.load_gather(xz_v, [ivs])
            DT[b][sl] = dx
            DT[b][pl.ds(CH + k * L, L)] = dy
            DT[b][pl.ds(2 * CH + k * L, L)] = dz
            D2[b][sl] = dx * dx + dy * dy + dz * dz
            return kcarry

        lax.fori_loop(0, CH // L, dist, 0, unroll=True)

    def rowadd_loop(b):
        def row(i, rcarry):
            for j in range(DH // L):
                sl = pl.ds(j * L, L)
                RA[b][i, sl] = RA[b][i, sl] + RB[b][i, sl]
            return rcarry

        lax.fori_loop(0, CH, row, 0, unroll=False)

    pltpu.sync_copy(xx_hbm, xx_v)
    pltpu.sync_copy(xy_hbm, xy_v)
    pltpu.sync_copy(xz_hbm, xz_v)
    issue_idx(0, 0)
    issue_idx(1, 1)
    issue_idx(2, 2)
    wait_idx(0)
    issue_gather(0)
    wait_idx(1)
    issue_gather(1)

    def body(i, carry):
        c = 3 * i
        dist_loop(0)
        wait_gather(0)
        issue_idx(0, c + 3)
        wait_idx(2)

        @pl.when(i > 0)
        def _():
            wait_writes(2)

        issue_gather(2)
        rowadd_loop(0)
        issue_writes(0, c)
        dist_loop(1)
        wait_gather(1)
        issue_idx(1, c + 4)
        wait_idx(0)
        wait_writes(0)
        issue_gather(0)
        rowadd_loop(1)
        issue_writes(1, c + 1)
        dist_loop(2)
        wait_gather(2)

        @pl.when(c + 5 < NCH)
        def _():
            issue_idx(2, c + 5)

        wait_idx(1)
        wait_writes(1)
        issue_gather(1)
        rowadd_loop(2)
        issue_writes(2, c + 2)
        return carry

    lax.fori_loop(0, (NCH - 2) // 3, body, 0, unroll=False)
    dist_loop(0)
    wait_gather(0)
    rowadd_loop(0)
    issue_writes(0, NCH - 2)
    dist_loop(1)
    wait_gather(1)
    rowadd_loop(1)
    issue_writes(1, NCH - 1)
    wait_writes(0)
    wait_writes(1)
    wait_writes(2)


def _gather_call(ap, bp, xx, xy, xz, src, dst):
    k = functools.partial(
        pl.kernel,
        out_type=(
            jax.ShapeDtypeStruct((E, DH), F32),
            jax.ShapeDtypeStruct((E,), F32),
            jax.ShapeDtypeStruct((3 * E,), F32),
        ),
        mesh=_mesh(),
        scratch_types=(
            [pltpu.VMEM((CH,), jnp.int32) for _ in range(6)]
            + [pltpu.VMEM((CH, DH), F32) for _ in range(6)]
            + [pltpu.VMEM((CH,), F32) for _ in range(3)]
            + [pltpu.VMEM((3 * CH,), F32) for _ in range(3)]
            + [pltpu.VMEM((N,), F32) for _ in range(3)]
            + [pltpu.SemaphoreType.DMA for _ in range(9)]
        ),
        compiler_params=pltpu.CompilerParams(needs_layout_passes=False),
    )(_gather_body)
    return k(ap, bp, xx, xy, xz, src, dst)


def _edge_body(s_ref, d2_ref, bt_ref, bW1, bb1, bW2, bb2, We, wc, cW,
               eW1, eb1, eW2, eb2, msg_ref, le_ref, sc_ref):
    dn0 = (((0,), (0,)), ((), ()))
    bt = bt_ref[0]
    e = jax.nn.silu(
        lax.dot_general(bt, bW1[...], dn0, preferred_element_type=F32) + bb1[...]
    )
    e = jnp.dot(e, bW2[...], preferred_element_type=F32) + bb2[...]
    d2row = d2_ref[0]
    d2wc = lax.dot_general(d2row, wc[...], dn0, preferred_element_type=F32)
    msg = jax.nn.silu(
        s_ref[...] + d2wc + jnp.dot(e, We[...], preferred_element_type=F32)
    )
    msg_ref[...] = msg
    z2 = jax.nn.silu(
        jnp.dot(msg, eW1[...], preferred_element_type=F32) + eb1[...]
    )
    le_ref[...] = jnp.dot(z2, eW2[...], preferred_element_type=F32) + eb2[...]
    dn_cw = (((0,), (1,)), ((), ()))
    sc_ref[0] = lax.dot_general(cW[...], msg, dn_cw, preferred_element_type=F32)


def _edge_call(S, d23, bt3, bW1, bb1, bW2, bb2, We, wc, cW, eW1, eb1, eW2, eb2):
    out_shape = (
        jax.ShapeDtypeStruct((E, DH), F32),
        jax.ShapeDtypeStruct((E, 5), F32),
        jax.ShapeDtypeStruct((NBLK, 1, EB), F32),
    )
    return pl.pallas_call(
        _edge_body,
        grid=(NBLK,),
        in_specs=[
            pl.BlockSpec((EB, DH), lambda i: (i, 0)),
            pl.BlockSpec((1, 1, EB), lambda i: (i, 0, 0)),
            pl.BlockSpec((1, 5, EB), lambda i: (i, 0, 0)),
            _full(bW1.shape), _full(bb1.shape), _full(bW2.shape), _full(bb2.shape),
            _full(We.shape), _full(wc.shape), _full(cW.shape),
            _full(eW1.shape), _full(eb1.shape), _full(eW2.shape), _full(eb2.shape),
        ],
        out_specs=(
            pl.BlockSpec((EB, DH), lambda i: (i, 0)),
            pl.BlockSpec((EB, 5), lambda i: (i, 0)),
            pl.BlockSpec((1, 1, EB), lambda i: (i, 0, 0)),
        ),
        out_shape=out_shape,
    )(S, d23, bt3, bW1, bb1, bW2, bb2, We, wc, cW, eW1, eb1, eW2, eb2)


def _aggscatter_body(msg_hbm, dst_hbm, z128_hbm, agg_hbm,
                     di0, di1, di2, m0, m1, m2,
                     ls0, ls1, ls2, ss0, ss1, ss2, agg_s):
    DI = (di0, di1, di2)
    M = (m0, m1, m2)
    LS = (ls0, ls1, ls2)
    SS = (ss0, ss1, ss2)
    c = lax.axis_index("c")
    s = lax.axis_index("s")
    wid = c * NS + s

    @pl.when(s < NS - 1)
    def _():
        sl = pl.ds(s * RPS, RPS)
        pltpu.sync_copy(z128_hbm.at[sl], agg_s.at[sl])

    @pl.when(s == NS - 1)
    def _():
        sl = pl.ds(RPS * (NS - 1), RPS_LAST)
        pltpu.sync_copy(z128_hbm.at[sl], agg_s.at[sl])

    plsc.subcore_barrier()
    base = wid * EPW

    def issue_loads(b, ci):
        off = base + ci * CH
        pltpu.async_copy(dst_hbm.at[pl.ds(off, CH)], DI[b], LS[b])
        pltpu.async_copy(msg_hbm.at[pl.ds(off, CH)], M[b], LS[b])

    def wait_loads(b):
        pltpu.make_async_copy(dst_hbm.at[pl.ds(0, CH)], DI[b], LS[b]).wait()
        pltpu.make_async_copy(msg_hbm.at[pl.ds(0, CH)], M[b], LS[b]).wait()

    def issue_scatter(b):
        pltpu.async_copy(M[b], agg_s.at[DI[b]], SS[b], add=True)

    def wait_scatter(b):
        pltpu.make_async_copy(M[b], agg_s.at[pl.ds(0, CH)], SS[b]).wait()

    issue_loads(0, 0)
    issue_loads(1, 1)

    def body(i, carry):
        c = 3 * i
        wait_loads(0)
        issue_scatter(0)

        @pl.when(i > 0)
        def _():
            wait_scatter(2)

        issue_loads(2, c + 2)
        wait_loads(1)
        issue_scatter(1)
        wait_scatter(0)
        issue_loads(0, c + 3)
        wait_loads(2)
        issue_scatter(2)
        wait_scatter(1)
        issue_loads(1, c + 4)
        return carry

    lax.fori_loop(0, (NCH - 2) // 3, body, 0, unroll=False)
    wait_scatter(2)
    wait_loads(0)
    issue_scatter(0)
    wait_loads(1)
    issue_scatter(1)
    wait_scatter(0)
    wait_scatter(1)
    plsc.subcore_barrier()

    @pl.when(s < NS - 1)
    def _():
        sl = pl.ds(s * RPS, RPS)
        pltpu.sync_copy(agg_s.at[sl], agg_hbm.at[c, sl])

    @pl.when(s == NS - 1)
    def _():
        sl = pl.ds(RPS * (NS - 1), RPS_LAST)
        pltpu.sync_copy(agg_s.at[sl], agg_hbm.at[c, sl])


def _velscatter_body(scal_hbm, dst_hbm, dt_hbm, zn_hbm,
                     vx_hbm, vy_hbm, vz_hbm,
                     di0, di1, di2, sc0, sc1, sc2, dtc0, dtc1, dtc2,
                     vx_v, vy_v, vz_v, ls0, ls1, ls2):
    DI = (di0, di1, di2)
    SC = (sc0, sc1, sc2)
    DTC = (dtc0, dtc1, dtc2)
    LS = (ls0, ls1, ls2)
    c = lax.axis_index("c")
    s = lax.axis_index("s")
    wid = c * NS + s
    pltpu.sync_copy(zn_hbm, vx_v)
    pltpu.sync_copy(zn_hbm, vy_v)
    pltpu.sync_copy(zn_hbm, vz_v)
    base = wid * EPW

    def issue_loads(b, ci):
        off = base + ci * CH
        pltpu.async_copy(dst_hbm.at[pl.ds(off, CH)], DI[b], LS[b])
        pltpu.async_copy(scal_hbm.at[pl.ds(off, CH)], SC[b], LS[b])
        pltpu.async_copy(dt_hbm.at[pl.ds(3 * off, 3 * CH)], DTC[b], LS[b])

    def wait_loads(b):
        pltpu.make_async_copy(dst_hbm.at[pl.ds(0, CH)], DI[b], LS[b]).wait()
        pltpu.make_async_copy(scal_hbm.at[pl.ds(0, CH)], SC[b], LS[b]).wait()
        pltpu.make_async_copy(dt_hbm.at[pl.ds(0, 3 * CH)], DTC[b], LS[b]).wait()

    def vel_loop(b):
        def vel(k, kcarry):
            ksl = pl.ds(k * L, L)
            ivd = DI[b][ksl]
            scv = SC[b][ksl]
            plsc.addupdate_scatter(vx_v, [ivd], DTC[b][ksl] * scv)
            plsc.addupdate_scatter(vy_v, [ivd],
                                   DTC[b][pl.ds(CH + k * L, L)] * scv)
            plsc.addupdate_scatter(vz_v, [ivd],
                                   DTC[b][pl.ds(2 * CH + k * L, L)] * scv)
            return kcarry

        lax.fori_loop(0, CH // L, vel, 0, unroll=True)

    issue_loads(0, 0)
    issue_loads(1, 1)

    def body(i, carry):
        c = 3 * i
        issue_loads(2, c + 2)
        wait_loads(0)
        vel_loop(0)
        issue_loads(0, c + 3)
        wait_loads(1)
        vel_loop(1)
        issue_loads(1, c + 4)
        wait_loads(2)
        vel_loop(2)
        return carry

    lax.fori_loop(0, (NCH - 2) // 3, body, 0, unroll=False)
    wait_loads(0)
    vel_loop(0)
    wait_loads(1)
    vel_loop(1)
    pltpu.sync_copy(vx_v, vx_hbm.at[wid, 0])
    pltpu.sync_copy(vy_v, vy_hbm.at[wid, 0])
    pltpu.sync_copy(vz_v, vz_hbm.at[wid, 0])


def _scatter_call(msg, scal, dst, dtri):
    z128 = jnp.zeros((N, DH), F32)
    zn = jnp.zeros((N,), F32)
    ka = functools.partial(
        pl.kernel,
        out_type=jax.ShapeDtypeStruct((NC, N, DH), F32),
        mesh=_mesh(),
        scratch_types=(
            [pltpu.VMEM((CH,), jnp.int32) for _ in range(3)]
            + [pltpu.VMEM((CH, DH), F32) for _ in range(3)]
            + [pltpu.SemaphoreType.DMA for _ in range(6)]
            + [pltpu.VMEM_SHARED((N, DH), F32)]
        ),
        compiler_params=pltpu.CompilerParams(needs_layout_passes=False),
    )(_aggscatter_body)
    aggp = ka(msg, dst, z128)
    kv = functools.partial(
        pl.kernel,
        out_type=(
            jax.ShapeDtypeStruct((NW, 1, N), F32),
            jax.ShapeDtypeStruct((NW, 1, N), F32),
            jax.ShapeDtypeStruct((NW, 1, N), F32),
        ),
        mesh=_mesh(),
        scratch_types=(
            [pltpu.VMEM((CH,), jnp.int32) for _ in range(3)]
            + [pltpu.VMEM((CH,), F32) for _ in range(3)]
            + [pltpu.VMEM((3 * CH,), F32) for _ in range(3)]
            + [pltpu.VMEM((N,), F32) for _ in range(3)]
            + [pltpu.SemaphoreType.DMA for _ in range(3)]
        ),
        compiler_params=pltpu.CompilerParams(needs_layout_passes=False),
    )(_velscatter_body)
    vx, vy, vz = kv(scal, dst, dtri, zn)
    return aggp, vx, vy, vz


def _dec_body(h_ref, agg_ref, vx_ref, vy_ref, vz_ref, updW, W1, b1, W2, b2,
              vel_ref, lh_ref):
    agg = agg_ref[0] + agg_ref[1]
    h_out = h_ref[...] + jnp.dot(agg, updW[...], preferred_element_type=F32)
    z = jax.nn.silu(
        jnp.dot(h_out, W1[...], preferred_element_type=F32) + b1[...]
    )
    lh_ref[...] = jnp.dot(z, W2[...], preferred_element_type=F32) + b2[...]
    cx = jnp.sum(vx_ref[...], axis=1, keepdims=True)
    cy = jnp.sum(vy_ref[...], axis=1, keepdims=True)
    cz = jnp.sum(vz_ref[...], axis=1, keepdims=True)
    vel_ref[...] = jnp.concatenate([cx, cy, cz], axis=1)


def _dec_call(h, aggp, vxT, vyT, vzT, updW, W1, b1, W2, b2):
    out_shape = (
        jax.ShapeDtypeStruct((N, 3), F32),
        jax.ShapeDtypeStruct((N, DH), F32),
    )
    return pl.pallas_call(
        _dec_body,
        grid=(N // NB,),
        in_specs=[
            pl.BlockSpec((NB, DH), lambda i: (i, 0)),
            pl.BlockSpec((NC, NB, DH), lambda i: (0, i, 0)),
            pl.BlockSpec((NB, NW), lambda i: (i, 0)),
            pl.BlockSpec((NB, NW), lambda i: (i, 0)),
            pl.BlockSpec((NB, NW), lambda i: (i, 0)),
            _full(updW.shape),
            _full(W1.shape), _full(b1.shape), _full(W2.shape), _full(b2.shape),
        ],
        out_specs=(
            pl.BlockSpec((NB, 3), lambda i: (i, 0)),
            pl.BlockSpec((NB, DH), lambda i: (i, 0)),
        ),
        out_shape=out_shape,
    )(h, aggp, vxT, vyT, vzT, updW, W1, b1, W2, b2)


def kernel(x_atoms, h_atoms, rot_vec, mask_atoms, pocket_h, mask_pocket, t,
           bond_index, bond_types, params):
    p = params
    r1 = lambda b: b.reshape(1, -1)

    maskp_f = mask_pocket.astype(F32).reshape(NP_, 1)
    maska_f = mask_atoms.astype(F32).reshape(N, 1)
    xx = x_atoms[:, 0]
    xy = x_atoms[:, 1]
    xz = x_atoms[:, 2]
    src = bond_index[0].astype(jnp.int32)
    dst = bond_index[1].astype(jnp.int32)
    bt3 = bond_types.reshape(NBLK, EB, 5).transpose(0, 2, 1)

    msg_W = p['msg_W']
    Wsrc = msg_W[0:DH]
    Wdst = msg_W[DH:2 * DH]
    wc = msg_W[2 * DH:2 * DH + 1]
    We = msg_W[2 * DH + 1:]

    pool = _pool_call(pocket_h, maskp_f,
                      p['re_W1'], r1(p['re_b1']), p['re_W2'], r1(p['re_b2']))
    h, ap, bp = _node_call(h_atoms, maska_f, pool, t,
                           p['ae_W1'], r1(p['ae_b1']), p['ae_W2'], r1(p['ae_b2']),
                           Wsrc, Wdst, r1(p['msg_b']))
    S, d2, dtri = _gather_call(ap, bp, xx, xy, xz, src, dst)
    msg, logits_e, scal3 = _edge_call(
        S, d2.reshape(NBLK, 1, EB), bt3,
        p['be_W1'], r1(p['be_b1']), p['be_W2'], r1(p['be_b2']),
        We, wc, p['cW'],
        p['ed_W1'], r1(p['ed_b1']), p['ed_W2'], r1(p['ed_b2']))
    aggp, vx, vy, vz = _scatter_call(msg, scal3.reshape(E), dst, dtri)
    vx = vx.reshape(NW, N).T
    vy = vy.reshape(NW, N).T
    vz = vz.reshape(NW, N).T
    vel, logits_h = _dec_call(h, aggp, vx, vy, vz, p['upd_W'],
                              p['ad_W1'], r1(p['ad_b1']),
                              p['ad_W2'], r1(p['ad_b2']))
    return vel, logits_h, logits_e


# speedup vs baseline: 1.3820x; 1.0034x over previous
import functools

import jax
import jax.numpy as jnp
from jax import lax
from jax.experimental import pallas as pl
from jax.experimental.pallas import tpu as pltpu
from jax.experimental.pallas import tpu_sc as plsc

F32 = jnp.float32

N = 10000
NP_ = 10000
E = 320000
NG = 16
DH = 128

NC = 2
NS = 16
L = 16
NW = NC * NS
EPW = E // NW
CH = 80
NCH = EPW // CH
RPS = 632
RPS_LAST = N - RPS * (NS - 1)

NB = 2000
EB = 8000
NBLK = E // EB


def _full(shape):
    return pl.BlockSpec(shape, lambda i: tuple(0 for _ in shape))


def _mesh():
    return plsc.VectorSubcoreMesh(
        core_axis_name="c", subcore_axis_name="s", num_cores=NC, num_subcores=NS
    )


def _pool_body(hp_ref, mk_ref, W1, b1, W2, b2, out_ref, acc, cnt):
    i = pl.program_id(0)

    @pl.when(i == 0)
    def _():
        acc[...] = jnp.zeros_like(acc)
        cnt[...] = jnp.zeros_like(cnt)

    z = jax.nn.silu(
        jnp.dot(hp_ref[...], W1[...], preferred_element_type=F32) + b1[...]
    )
    z = jnp.dot(z, W2[...], preferred_element_type=F32) + b2[...]
    gid = lax.broadcasted_iota(jnp.int32, (NB, NG), 1).astype(F32)
    onehot = (mk_ref[...] == gid).astype(F32)
    dn = (((0,), (0,)), ((), ()))
    acc[...] += lax.dot_general(onehot, z, dn, preferred_element_type=F32)
    ones = jnp.ones((NB, 1), F32)
    cnt[...] += lax.dot_general(onehot, ones, dn, preferred_element_type=F32)

    @pl.when(i == pl.num_programs(0) - 1)
    def _():
        out_ref[...] = acc[...] / jnp.maximum(cnt[...], 1.0)


def _pool_call(pocket_h, maskp_f, W1, b1, W2, b2):
    return pl.pallas_call(
        _pool_body,
        grid=(NP_ // NB,),
        in_specs=[
            pl.BlockSpec((NB, DH), lambda i: (i, 0)),
            pl.BlockSpec((NB, 1), lambda i: (i, 0)),
            _full(W1.shape), _full(b1.shape), _full(W2.shape), _full(b2.shape),
        ],
        out_specs=_full((NG, DH)),
        out_shape=jax.ShapeDtypeStruct((NG, DH), F32),
        scratch_shapes=[pltpu.VMEM((NG, DH), F32), pltpu.VMEM((NG, 1), F32)],
    )(pocket_h, maskp_f, W1, b1, W2, b2)


def _node_body(ha_ref, mk_ref, pool_ref, t_ref,
               W1, b1, W2, b2, Wsrc, Wdst, msg_b,
               h_ref, ap_ref, bp_ref):
    z = jax.nn.silu(
        jnp.dot(ha_ref[...], W1[...], preferred_element_type=F32) + b1[...]
    )
    z = jnp.dot(z, W2[...], preferred_element_type=F32) + b2[...]
    gid = lax.broadcasted_iota(jnp.int32, (NB, NG), 1).astype(F32)
    onehot = (mk_ref[...] == gid).astype(F32)
    z = z + jnp.dot(onehot, t_ref[...], preferred_element_type=F32)
    z = z + jnp.dot(onehot, pool_ref[...], preferred_element_type=F32)
    h_ref[...] = z
    ap_ref[...] = jnp.dot(z, Wsrc[...], preferred_element_type=F32)
    bp_ref[...] = jnp.dot(z, Wdst[...], preferred_element_type=F32) + msg_b[...]


def _node_call(h_atoms, maska_f, pool, t, W1, b1, W2, b2, Wsrc, Wdst, msg_b):
    out_shape = tuple(jax.ShapeDtypeStruct((N, DH), F32) for _ in range(3))
    return pl.pallas_call(
        _node_body,
        grid=(N // NB,),
        in_specs=[
            pl.BlockSpec((NB, DH), lambda i: (i, 0)),
            pl.BlockSpec((NB, 1), lambda i: (i, 0)),
            _full((NG, DH)), _full((NG, 1)),
            _full(W1.shape), _full(b1.shape), _full(W2.shape), _full(b2.shape),
            _full(Wsrc.shape), _full(Wdst.shape), _full(msg_b.shape),
        ],
        out_specs=tuple(pl.BlockSpec((NB, DH), lambda i: (i, 0)) for _ in range(3)),
        out_shape=out_shape,
    )(h_atoms, maska_f, pool, t, W1, b1, W2, b2, Wsrc, Wdst, msg_b)


def _gather_body(ap_hbm, bp_hbm, xx_hbm, xy_hbm, xz_hbm, src_hbm, dst_hbm,
                 s_hbm, d2_hbm, dt_hbm,
                 si0, si1, si2, di0, di1, di2, ra0, ra1, ra2,
                 rb0, rb1, rb2, d20, d21, d22, dt0, dt1, dt2,
                 xx_v, xy_v, xz_v,
                 is0, is1, is2, gs0, gs1, gs2, ws0, ws1, ws2):
    SI = (si0, si1, si2)
    DI = (di0, di1, di2)
    RA = (ra0, ra1, ra2)
    RB = (rb0, rb1, rb2)
    D2 = (d20, d21, d22)
    DT = (dt0, dt1, dt2)
    IS = (is0, is1, is2)
    GS = (gs0, gs1, gs2)
    WS = (ws0, ws1, ws2)
    wid = lax.axis_index("s") * NC + lax.axis_index("c")
    base = wid * EPW

    def issue_idx(b, c):
        off = base + c * CH
        pltpu.async_copy(src_hbm.at[pl.ds(off, CH)], SI[b], IS[b])
        pltpu.async_copy(dst_hbm.at[pl.ds(off, CH)], DI[b], IS[b])

    def wait_idx(b):
        pltpu.make_async_copy(src_hbm.at[pl.ds(0, CH)], SI[b], IS[b]).wait()
        pltpu.make_async_copy(src_hbm.at[pl.ds(0, CH)], DI[b], IS[b]).wait()

    def issue_gather(b):
        pltpu.async_copy(ap_hbm.at[SI[b]], RA[b], GS[b])
        pltpu.async_copy(bp_hbm.at[DI[b]], RB[b], GS[b])

    def wait_gather(b):
        pltpu.make_async_copy(ap_hbm.at[pl.ds(0, CH)], RA[b], GS[b]).wait()
        pltpu.make_async_copy(bp_hbm.at[pl.ds(0, CH)], RB[b], GS[b]).wait()

    def issue_writes(b, c):
        off = base + c * CH
        pltpu.async_copy(RA[b], s_hbm.at[pl.ds(off, CH)], WS[b])
        pltpu.async_copy(D2[b], d2_hbm.at[pl.ds(off, CH)], WS[b])
        pltpu.async_copy(DT[b], dt_hbm.at[pl.ds(3 * off, 3 * CH)], WS[b])

    def wait_writes(b):
        pltpu.make_async_copy(RA[b], s_hbm.at[pl.ds(0, CH)], WS[b]).wait()
        pltpu.make_async_copy(D2[b], d2_hbm.at[pl.ds(0, CH)], WS[b]).wait()
        pltpu.make_async_copy(DT[b], dt_hbm.at[pl.ds(0, 3 * CH)], WS[b]).wait()

    def dist_loop(b):
        def dist(k, kcarry):
            sl = pl.ds(k * L, L)
            ivs = SI[b][sl]
            ivd = DI[b][sl]
            dx = plsc.load_gather(xx_v, [ivd]) - plsc.load_gather(xx_v, [ivs])
            dy = plsc.load_gather(xy_v, [ivd]) - plsc.load_gather(xy_v, [ivs])
            dz = plsc.load_gather(xz_v, [ivd]) - plsc.load_gather(xz_v, [ivs])
            DT[b][sl] = dx
            DT[b][pl.ds(CH + k * L, L)] = dy
            DT[b][pl.ds(2 * CH + k * L, L)] = dz
            D2[b][sl] = dx * dx + dy * dy + dz * dz
            return kcarry

        lax.fori_loop(0, CH // L, dist, 0, unroll=True)

    def rowadd_loop(b):
        def row(i, rcarry):
            for j in range(DH // L):
                sl = pl.ds(j * L, L)
                RA[b][i, sl] = RA[b][i, sl] + RB[b][i, sl]
            return rcarry

        lax.fori_loop(0, CH, row, 0, unroll=False)

    pltpu.sync_copy(xx_hbm, xx_v)
    pltpu.sync_copy(xy_hbm, xy_v)
    pltpu.sync_copy(xz_hbm, xz_v)
    issue_idx(0, 0)
    issue_idx(1, 1)
    issue_idx(2, 2)
    wait_idx(0)
    issue_gather(0)
    wait_idx(1)
    issue_gather(1)

    def body(i, carry):
        c = 3 * i
        dist_loop(0)
        wait_gather(0)
        issue_idx(0, c + 3)
        wait_idx(2)

        @pl.when(i > 0)
        def _():
            wait_writes(2)

        issue_gather(2)
        rowadd_loop(0)
        issue_writes(0, c)
        dist_loop(1)
        wait_gather(1)
        issue_idx(1, c + 4)
        wait_idx(0)
        wait_writes(0)
        issue_gather(0)
        rowadd_loop(1)
        issue_writes(1, c + 1)
        dist_loop(2)
        wait_gather(2)

        @pl.when(c + 5 < NCH)
        def _():
            issue_idx(2, c + 5)

        wait_idx(1)
        wait_writes(1)
        issue_gather(1)
        rowadd_loop(2)
        issue_writes(2, c + 2)
        return carry

    lax.fori_loop(0, (NCH - 2) // 3, body, 0, unroll=False)
    dist_loop(0)
    wait_gather(0)
    rowadd_loop(0)
    issue_writes(0, NCH - 2)
    dist_loop(1)
    wait_gather(1)
    rowadd_loop(1)
    issue_writes(1, NCH - 1)
    wait_writes(0)
    wait_writes(1)
    wait_writes(2)


def _gather_call(ap, bp, xx, xy, xz, src, dst):
    k = functools.partial(
        pl.kernel,
        out_type=(
            jax.ShapeDtypeStruct((E, DH), F32),
            jax.ShapeDtypeStruct((E,), F32),
            jax.ShapeDtypeStruct((3 * E,), F32),
        ),
        mesh=_mesh(),
        scratch_types=(
            [pltpu.VMEM((CH,), jnp.int32) for _ in range(6)]
            + [pltpu.VMEM((CH, DH), F32) for _ in range(6)]
            + [pltpu.VMEM((CH,), F32) for _ in range(3)]
            + [pltpu.VMEM((3 * CH,), F32) for _ in range(3)]
            + [pltpu.VMEM((N,), F32) for _ in range(3)]
            + [pltpu.SemaphoreType.DMA for _ in range(9)]
        ),
        compiler_params=pltpu.CompilerParams(needs_layout_passes=False),
    )(_gather_body)
    return k(ap, bp, xx, xy, xz, src, dst)


def _edge_body(s_ref, d2_ref, bt_ref, bW1, bb1, bW2, bb2, We, wc, cW,
               eW1, eb1, eW2, eb2, msg_ref, le_ref, sc_ref):
    dn0 = (((0,), (0,)), ((), ()))
    bt = bt_ref[0]
    e = jax.nn.silu(
        lax.dot_general(bt, bW1[...], dn0, preferred_element_type=F32) + bb1[...]
    )
    e = jnp.dot(e, bW2[...], preferred_element_type=F32) + bb2[...]
    d2row = d2_ref[0]
    d2wc = lax.dot_general(d2row, wc[...], dn0, preferred_element_type=F32)
    msg = jax.nn.silu(
        s_ref[...] + d2wc + jnp.dot(e, We[...], preferred_element_type=F32)
    )
    msg_ref[...] = msg
    z2 = jax.nn.silu(
        jnp.dot(msg, eW1[...], preferred_element_type=F32) + eb1[...]
    )
    le_ref[...] = jnp.dot(z2, eW2[...], preferred_element_type=F32) + eb2[...]
    dn_cw = (((0,), (1,)), ((), ()))
    sc_ref[0] = lax.dot_general(cW[...], msg, dn_cw, preferred_element_type=F32)


def _edge_call(S, d23, bt3, bW1, bb1, bW2, bb2, We, wc, cW, eW1, eb1, eW2, eb2):
    out_shape = (
        jax.ShapeDtypeStruct((E, DH), F32),
        jax.ShapeDtypeStruct((E, 5), F32),
        jax.ShapeDtypeStruct((NBLK, 1, EB), F32),
    )
    return pl.pallas_call(
        _edge_body,
        grid=(NBLK,),
        in_specs=[
            pl.BlockSpec((EB, DH), lambda i: (i, 0)),
            pl.BlockSpec((1, 1, EB), lambda i: (i, 0, 0)),
            pl.BlockSpec((1, 5, EB), lambda i: (i, 0, 0)),
            _full(bW1.shape), _full(bb1.shape), _full(bW2.shape), _full(bb2.shape),
            _full(We.shape), _full(wc.shape), _full(cW.shape),
            _full(eW1.shape), _full(eb1.shape), _full(eW2.shape), _full(eb2.shape),
        ],
        out_specs=(
            pl.BlockSpec((EB, DH), lambda i: (i, 0)),
            pl.BlockSpec((EB, 5), lambda i: (i, 0)),
            pl.BlockSpec((1, 1, EB), lambda i: (i, 0, 0)),
        ),
        out_shape=out_shape,
    )(S, d23, bt3, bW1, bb1, bW2, bb2, We, wc, cW, eW1, eb1, eW2, eb2)


def _aggscatter_body(msg_hbm, dst_hbm, z128_hbm, agg_hbm,
                     di0, di1, di2, di3, m0, m1, m2, m3,
                     ls0, ls1, ls2, ls3, ss0, ss1, ss2, ss3, agg_s):
    DI = (di0, di1, di2, di3)
    M = (m0, m1, m2, m3)
    LS = (ls0, ls1, ls2, ls3)
    SS = (ss0, ss1, ss2, ss3)
    c = lax.axis_index("c")
    s = lax.axis_index("s")
    wid = c * NS + s

    @pl.when(s < NS - 1)
    def _():
        sl = pl.ds(s * RPS, RPS)
        pltpu.sync_copy(z128_hbm.at[sl], agg_s.at[sl])

    @pl.when(s == NS - 1)
    def _():
        sl = pl.ds(RPS * (NS - 1), RPS_LAST)
        pltpu.sync_copy(z128_hbm.at[sl], agg_s.at[sl])

    plsc.subcore_barrier()
    base = wid * EPW

    def issue_loads(b, ci):
        off = base + ci * CH
        pltpu.async_copy(dst_hbm.at[pl.ds(off, CH)], DI[b], LS[b])
        pltpu.async_copy(msg_hbm.at[pl.ds(off, CH)], M[b], LS[b])

    def wait_loads(b):
        pltpu.make_async_copy(dst_hbm.at[pl.ds(0, CH)], DI[b], LS[b]).wait()
        pltpu.make_async_copy(msg_hbm.at[pl.ds(0, CH)], M[b], LS[b]).wait()

    def issue_scatter(b):
        pltpu.async_copy(M[b], agg_s.at[DI[b]], SS[b], add=True)

    def wait_scatter(b):
        pltpu.make_async_copy(M[b], agg_s.at[pl.ds(0, CH)], SS[b]).wait()

    issue_loads(0, 0)
    issue_loads(1, 1)
    issue_loads(2, 2)

    def body(i, carry):
        c = 4 * i
        wait_loads(0)
        issue_scatter(0)

        @pl.when(i > 0)
        def _():
            wait_scatter(3)

        issue_loads(3, c + 3)
        wait_loads(1)
        issue_scatter(1)
        wait_scatter(0)
        issue_loads(0, c + 4)
        wait_loads(2)
        issue_scatter(2)
        wait_scatter(1)

        @pl.when(c + 5 < NCH)
        def _():
            issue_loads(1, c + 5)

        wait_loads(3)
        issue_scatter(3)
        wait_scatter(2)

        @pl.when(c + 6 < NCH)
        def _():
            issue_loads(2, c + 6)

        return carry

    lax.fori_loop(0, (NCH - 1) // 4, body, 0, unroll=False)
    wait_scatter(3)
    wait_loads(0)
    issue_scatter(0)
    wait_scatter(0)
    plsc.subcore_barrier()

    @pl.when(s < NS - 1)
    def _():
        sl = pl.ds(s * RPS, RPS)
        pltpu.sync_copy(agg_s.at[sl], agg_hbm.at[c, sl])

    @pl.when(s == NS - 1)
    def _():
        sl = pl.ds(RPS * (NS - 1), RPS_LAST)
        pltpu.sync_copy(agg_s.at[sl], agg_hbm.at[c, sl])


def _velscatter_body(scal_hbm, dst_hbm, dt_hbm, zn_hbm,
                     vx_hbm, vy_hbm, vz_hbm,
                     di0, di1, di2, sc0, sc1, sc2, dtc0, dtc1, dtc2,
                     vx_v, vy_v, vz_v, ls0, ls1, ls2):
    DI = (di0, di1, di2)
    SC = (sc0, sc1, sc2)
    DTC = (dtc0, dtc1, dtc2)
    LS = (ls0, ls1, ls2)
    c = lax.axis_index("c")
    s = lax.axis_index("s")
    wid = c * NS + s
    pltpu.sync_copy(zn_hbm, vx_v)
    pltpu.sync_copy(zn_hbm, vy_v)
    pltpu.sync_copy(zn_hbm, vz_v)
    base = wid * EPW

    def issue_loads(b, ci):
        off = base + ci * CH
        pltpu.async_copy(dst_hbm.at[pl.ds(off, CH)], DI[b], LS[b])
        pltpu.async_copy(scal_hbm.at[pl.ds(off, CH)], SC[b], LS[b])
        pltpu.async_copy(dt_hbm.at[pl.ds(3 * off, 3 * CH)], DTC[b], LS[b])

    def wait_loads(b):
        pltpu.make_async_copy(dst_hbm.at[pl.ds(0, CH)], DI[b], LS[b]).wait()
        pltpu.make_async_copy(scal_hbm.at[pl.ds(0, CH)], SC[b], LS[b]).wait()
        pltpu.make_async_copy(dt_hbm.at[pl.ds(0, 3 * CH)], DTC[b], LS[b]).wait()

    def vel_loop(b):
        def vel(k, kcarry):
            ksl = pl.ds(k * L, L)
            ivd = DI[b][ksl]
            scv = SC[b][ksl]
            plsc.addupdate_scatter(vx_v, [ivd], DTC[b][ksl] * scv)
            plsc.addupdate_scatter(vy_v, [ivd],
                                   DTC[b][pl.ds(CH + k * L, L)] * scv)
            plsc.addupdate_scatter(vz_v, [ivd],
                                   DTC[b][pl.ds(2 * CH + k * L, L)] * scv)
            return kcarry

        lax.fori_loop(0, CH // L, vel, 0, unroll=True)

    issue_loads(0, 0)
    issue_loads(1, 1)

    def body(i, carry):
        c = 3 * i
        issue_loads(2, c + 2)
        wait_loads(0)
        vel_loop(0)
        issue_loads(0, c + 3)
        wait_loads(1)
        vel_loop(1)
        issue_loads(1, c + 4)
        wait_loads(2)
        vel_loop(2)
        return carry

    lax.fori_loop(0, (NCH - 2) // 3, body, 0, unroll=False)
    wait_loads(0)
    vel_loop(0)
    wait_loads(1)
    vel_loop(1)
    pltpu.sync_copy(vx_v, vx_hbm.at[wid, 0])
    pltpu.sync_copy(vy_v, vy_hbm.at[wid, 0])
    pltpu.sync_copy(vz_v, vz_hbm.at[wid, 0])


def _scatter_call(msg, scal, dst, dtri):
    z128 = jnp.zeros((N, DH), F32)
    zn = jnp.zeros((N,), F32)
    ka = functools.partial(
        pl.kernel,
        out_type=jax.ShapeDtypeStruct((NC, N, DH), F32),
        mesh=_mesh(),
        scratch_types=(
            [pltpu.VMEM((CH,), jnp.int32) for _ in range(4)]
            + [pltpu.VMEM((CH, DH), F32) for _ in range(4)]
            + [pltpu.SemaphoreType.DMA for _ in range(8)]
            + [pltpu.VMEM_SHARED((N, DH), F32)]
        ),
        compiler_params=pltpu.CompilerParams(needs_layout_passes=False),
    )(_aggscatter_body)
    aggp = ka(msg, dst, z128)
    kv = functools.partial(
        pl.kernel,
        out_type=(
            jax.ShapeDtypeStruct((NW, 1, N), F32),
            jax.ShapeDtypeStruct((NW, 1, N), F32),
            jax.ShapeDtypeStruct((NW, 1, N), F32),
        ),
        mesh=_mesh(),
        scratch_types=(
            [pltpu.VMEM((CH,), jnp.int32) for _ in range(3)]
            + [pltpu.VMEM((CH,), F32) for _ in range(3)]
            + [pltpu.VMEM((3 * CH,), F32) for _ in range(3)]
            + [pltpu.VMEM((N,), F32) for _ in range(3)]
            + [pltpu.SemaphoreType.DMA for _ in range(3)]
        ),
        compiler_params=pltpu.CompilerParams(needs_layout_passes=False),
    )(_velscatter_body)
    vx, vy, vz = kv(scal, dst, dtri, zn)
    return aggp, vx, vy, vz


def _dec_body(h_ref, agg_ref, vx_ref, vy_ref, vz_ref, updW, W1, b1, W2, b2,
              vel_ref, lh_ref):
    agg = agg_ref[0] + agg_ref[1]
    h_out = h_ref[...] + jnp.dot(agg, updW[...], preferred_element_type=F32)
    z = jax.nn.silu(
        jnp.dot(h_out, W1[...], preferred_element_type=F32) + b1[...]
    )
    lh_ref[...] = jnp.dot(z, W2[...], preferred_element_type=F32) + b2[...]
    cx = jnp.sum(vx_ref[...], axis=1, keepdims=True)
    cy = jnp.sum(vy_ref[...], axis=1, keepdims=True)
    cz = jnp.sum(vz_ref[...], axis=1, keepdims=True)
    vel_ref[...] = jnp.concatenate([cx, cy, cz], axis=1)


def _dec_call(h, aggp, vxT, vyT, vzT, updW, W1, b1, W2, b2):
    out_shape = (
        jax.ShapeDtypeStruct((N, 3), F32),
        jax.ShapeDtypeStruct((N, DH), F32),
    )
    return pl.pallas_call(
        _dec_body,
        grid=(N // NB,),
        in_specs=[
            pl.BlockSpec((NB, DH), lambda i: (i, 0)),
            pl.BlockSpec((NC, NB, DH), lambda i: (0, i, 0)),
            pl.BlockSpec((NB, NW), lambda i: (i, 0)),
            pl.BlockSpec((NB, NW), lambda i: (i, 0)),
            pl.BlockSpec((NB, NW), lambda i: (i, 0)),
            _full(updW.shape),
            _full(W1.shape), _full(b1.shape), _full(W2.shape), _full(b2.shape),
        ],
        out_specs=(
            pl.BlockSpec((NB, 3), lambda i: (i, 0)),
            pl.BlockSpec((NB, DH), lambda i: (i, 0)),
        ),
        out_shape=out_shape,
    )(h, aggp, vxT, vyT, vzT, updW, W1, b1, W2, b2)


def kernel(x_atoms, h_atoms, rot_vec, mask_atoms, pocket_h, mask_pocket, t,
           bond_index, bond_types, params):
    p = params
    r1 = lambda b: b.reshape(1, -1)

    maskp_f = mask_pocket.astype(F32).reshape(NP_, 1)
    maska_f = mask_atoms.astype(F32).reshape(N, 1)
    xx = x_atoms[:, 0]
    xy = x_atoms[:, 1]
    xz = x_atoms[:, 2]
    src = bond_index[0].astype(jnp.int32)
    dst = bond_index[1].astype(jnp.int32)
    bt3 = bond_types.reshape(NBLK, EB, 5).transpose(0, 2, 1)

    msg_W = p['msg_W']
    Wsrc = msg_W[0:DH]
    Wdst = msg_W[DH:2 * DH]
    wc = msg_W[2 * DH:2 * DH + 1]
    We = msg_W[2 * DH + 1:]

    pool = _pool_call(pocket_h, maskp_f,
                      p['re_W1'], r1(p['re_b1']), p['re_W2'], r1(p['re_b2']))
    h, ap, bp = _node_call(h_atoms, maska_f, pool, t,
                           p['ae_W1'], r1(p['ae_b1']), p['ae_W2'], r1(p['ae_b2']),
                           Wsrc, Wdst, r1(p['msg_b']))
    S, d2, dtri = _gather_call(ap, bp, xx, xy, xz, src, dst)
    msg, logits_e, scal3 = _edge_call(
        S, d2.reshape(NBLK, 1, EB), bt3,
        p['be_W1'], r1(p['be_b1']), p['be_W2'], r1(p['be_b2']),
        We, wc, p['cW'],
        p['ed_W1'], r1(p['ed_b1']), p['ed_W2'], r1(p['ed_b2']))
    aggp, vx, vy, vz = _scatter_call(msg, scal3.reshape(E), dst, dtri)
    vx = vx.reshape(NW, N).T
    vy = vy.reshape(NW, N).T
    vz = vz.reshape(NW, N).T
    vel, logits_h = _dec_call(h, aggp, vx, vy, vz, p['upd_W'],
                              p['ad_W1'], r1(p['ad_b1']),
                              p['ad_W2'], r1(p['ad_b2']))
    return vel, logits_h, logits_e
